# baseline (speedup 1.0000x reference)
import functools

import jax
import jax.numpy as jnp
from jax import lax
from jax.experimental import pallas as pl
from jax.experimental.pallas import tpu as pltpu
from jax.experimental.pallas import tpu_sc as plsc

VOCAB = 100000
EMBED_DIM = 32
CONTEXT = 20
PAD_CTX = 32
FAN_IN = CONTEXT * EMBED_DIM
K = 7
VBS = 1024
VB = K * VBS
NBLK = (VOCAB + VB - 1) // VB
TAIL = VOCAB - (NBLK - 1) * VB


def _sc_gather(idx_pad, table):
    mesh = plsc.VectorSubcoreMesh(core_axis_name="c", subcore_axis_name="s")

    @functools.partial(
        pl.kernel,
        mesh=mesh,
        out_type=jax.ShapeDtypeStruct((PAD_CTX, EMBED_DIM), jnp.float32),
        scratch_types=[
            pltpu.VMEM((PAD_CTX,), jnp.int32),
            pltpu.VMEM((PAD_CTX, EMBED_DIM), jnp.float32),
            pltpu.SemaphoreType.DMA,
        ],
        compiler_params=pltpu.CompilerParams(use_tc_tiling_on_sc=False),
    )
    def k(idx_hbm, table_hbm, out_hbm, idx_v, rows_v, sem):
        wid = lax.axis_index("s") * 2 + lax.axis_index("c")

        @pl.when(wid == 0)
        def _():
            pltpu.sync_copy(idx_hbm, idx_v)
            pltpu.async_copy(table_hbm.at[idx_v], rows_v, sem).wait()
            pltpu.sync_copy(rows_v, out_hbm)

    return k(idx_pad, table)


def _matvec_body(*refs):
    e_ref = refs[0]
    w_refs = refs[1:1 + K]
    b_ref = refs[1 + K]
    out_ref = refs[2 + K]
    m_ref, s_ref = refs[3 + K], refs[4 + K]

    i = pl.program_id(0)

    @pl.when(i == 0)
    def _():
        m_ref[0, 0] = -jnp.inf
        s_ref[0, 0] = 0.0

    xs = [
        lax.dot_general(
            e_ref[...], w_ref[...], (((1,), (1,)), ((), ())),
            preferred_element_type=jnp.float32,
        )
        for w_ref in w_refs
    ]
    x = jnp.concatenate(xs, axis=1)
    x = jnp.maximum(x + b_ref[...].reshape(1, VB), 0.0)

    last = pl.num_programs(0) - 1

    @pl.when(i < last)
    def _():
        out_ref[:, pl.ds(i * VB, VB)] = x

    col = i * VB + lax.broadcasted_iota(jnp.int32, (1, VB), 1)
    xm = jnp.where(col < VOCAB, x, -jnp.inf)
    m_old = m_ref[0, 0]
    m_new = jnp.maximum(m_old, jnp.max(xm))
    s_ref[0, 0] = s_ref[0, 0] * jnp.exp(m_old - m_new) + jnp.sum(
        jnp.exp(xm - m_new))
    m_ref[0, 0] = m_new

    @pl.when(i == last)
    def _():
        out_ref[:, pl.ds(last * VB, TAIL)] = x[:, :TAIL]
        logz = m_ref[0, 0] + jnp.log(s_ref[0, 0])
        out_ref[...] = out_ref[...] - logz


def kernel(inputs, emb_table, W, b):
    idx = jnp.zeros((PAD_CTX,), jnp.int32).at[:CONTEXT].set(
        inputs.astype(jnp.int32))
    rows = _sc_gather(idx, emb_table)
    e = rows[:CONTEXT].reshape(1, FAN_IN)

    w_specs = [
        pl.BlockSpec((VBS, FAN_IN), functools.partial(
            lambda i, j: (K * i + j, 0), j=j))
        for j in range(K)
    ]
    out = pl.pallas_call(
        _matvec_body,
        grid=(NBLK,),
        in_specs=[pl.BlockSpec((1, FAN_IN), lambda i: (0, 0))]
        + w_specs
        + [pl.BlockSpec((VB,), lambda i: (i,))],
        out_specs=pl.BlockSpec((1, VOCAB), lambda i: (0, 0)),
        out_shape=jax.ShapeDtypeStruct((1, VOCAB), jnp.float32),
        scratch_shapes=[
            pltpu.SMEM((1, 1), jnp.float32),
            pltpu.SMEM((1, 1), jnp.float32),
        ],
        compiler_params=pltpu.CompilerParams(
            dimension_semantics=("arbitrary",)),
    )(e, *([W] * K), b)

    return out


# speedup vs baseline: 1.0004x; 1.0004x over previous
import functools

import jax
import jax.numpy as jnp
from jax import lax
from jax.experimental import pallas as pl
from jax.experimental.pallas import tpu as pltpu
from jax.experimental.pallas import tpu_sc as plsc

VOCAB = 100000
EMBED_DIM = 32
CONTEXT = 20
PAD_CTX = 32
FAN_IN = CONTEXT * EMBED_DIM
VBS = 2048
NFULL = VOCAB // VBS
TAIL = VOCAB - NFULL * VBS
NB = NFULL + 1
DEPTH = 4


def _sc_gather(idx_pad, table):
    mesh = plsc.VectorSubcoreMesh(core_axis_name="c", subcore_axis_name="s")

    @functools.partial(
        pl.kernel,
        mesh=mesh,
        out_type=jax.ShapeDtypeStruct((PAD_CTX, EMBED_DIM), jnp.float32),
        scratch_types=[
            pltpu.VMEM((PAD_CTX,), jnp.int32),
            pltpu.VMEM((PAD_CTX, EMBED_DIM), jnp.float32),
            pltpu.SemaphoreType.DMA,
        ],
        compiler_params=pltpu.CompilerParams(use_tc_tiling_on_sc=False),
    )
    def k(idx_hbm, table_hbm, out_hbm, idx_v, rows_v, sem):
        wid = lax.axis_index("s") * 2 + lax.axis_index("c")

        @pl.when(wid == 0)
        def _():
            pltpu.sync_copy(idx_hbm, idx_v)
            pltpu.async_copy(table_hbm.at[idx_v], rows_v, sem).wait()
            pltpu.sync_copy(rows_v, out_hbm)

    return k(idx_pad, table)


def _matvec_body(e_ref, w_hbm, b_ref, out_ref, w_buf, sems):
    def copy_for(blk, nrows):
        slot = lax.rem(blk, DEPTH)
        return pltpu.make_async_copy(
            w_hbm.at[pl.ds(blk * VBS, nrows), :],
            w_buf.at[slot, pl.ds(0, nrows), :],
            sems.at[slot],
        )

    for d in range(DEPTH):
        copy_for(d, VBS if d < NFULL else TAIL).start()

    def block_x(blk, nrows):
        slot = lax.rem(blk, DEPTH)
        w = w_buf[slot, pl.ds(0, nrows), :]
        x = lax.dot_general(
            e_ref[...], w, (((1,), (1,)), ((), ())),
            preferred_element_type=jnp.float32,
        )
        bb = b_ref[pl.ds(blk * VBS, nrows)].reshape(1, nrows)
        return jnp.maximum(x + bb, 0.0)

    def step(blk, carry):
        m, s = carry
        copy_for(blk, VBS).wait()
        x = block_x(blk, VBS)
        out_ref[:, pl.ds(blk * VBS, VBS)] = x

        @pl.when(blk + DEPTH < NB - 1)
        def _():
            copy_for(blk + DEPTH, VBS).start()

        @pl.when(blk + DEPTH == NB - 1)
        def _():
            copy_for(blk + DEPTH, TAIL).start()

        m_new = jnp.maximum(m, jnp.max(x))
        s_new = s * jnp.exp(m - m_new) + jnp.sum(jnp.exp(x - m_new))
        return m_new, s_new

    m, s = lax.fori_loop(0, NFULL, step, (-jnp.inf, 0.0))

    copy_for(NFULL, TAIL).wait()
    x = block_x(NFULL, TAIL)
    out_ref[:, pl.ds(NFULL * VBS, TAIL)] = x
    m_new = jnp.maximum(m, jnp.max(x))
    s = s * jnp.exp(m - m_new) + jnp.sum(jnp.exp(x - m_new))

    logz = m_new + jnp.log(s)
    out_ref[...] = out_ref[...] - logz


def kernel(inputs, emb_table, W, b):
    idx = jnp.zeros((PAD_CTX,), jnp.int32).at[:CONTEXT].set(
        inputs.astype(jnp.int32))
    rows = _sc_gather(idx, emb_table)
    e = rows[:CONTEXT].reshape(1, FAN_IN)

    out = pl.pallas_call(
        _matvec_body,
        in_specs=[
            pl.BlockSpec((1, FAN_IN), lambda: (0, 0)),
            pl.BlockSpec(memory_space=pl.ANY),
            pl.BlockSpec((VOCAB,), lambda: (0,)),
        ],
        out_specs=pl.BlockSpec((1, VOCAB), lambda: (0, 0)),
        out_shape=jax.ShapeDtypeStruct((1, VOCAB), jnp.float32),
        scratch_shapes=[
            pltpu.VMEM((DEPTH, VBS, FAN_IN), jnp.float32),
            pltpu.SemaphoreType.DMA((DEPTH,)),
        ],
    )(e, W, b)

    return out
